# baseline (speedup 1.0000x reference)
import functools

import jax
import jax.numpy as jnp
from jax import lax
from jax.experimental import pallas as pl
from jax.experimental.pallas import tpu as pltpu
from jax.experimental.pallas import tpu_sc as plsc

NUM_CORES = 2
NUM_SUBCORES = 16
NUM_WORKERS = NUM_CORES * NUM_SUBCORES
DIM = 64
NBUF = 8


def _make_kernel(n_tok: int, seq: int):
    mesh = plsc.VectorSubcoreMesh(core_axis_name="c", subcore_axis_name="s")
    tblk = n_tok // NUM_WORKERS

    @functools.partial(
        pl.kernel,
        out_type=jax.ShapeDtypeStruct((n_tok, seq, DIM), jnp.float32),
        mesh=mesh,
        scratch_types=[
            pltpu.VMEM((tblk, seq), jnp.int32),
            pltpu.VMEM((NBUF, seq, DIM), jnp.float32),
            pltpu.SemaphoreType.DMA,
            pltpu.SemaphoreType.DMA,
        ],
        compiler_params=pltpu.CompilerParams(use_tc_tiling_on_sc=False),
    )
    def k(x_hbm, w_hbm, out_hbm, xst, bufs, gsem, wsem):
        wid = lax.axis_index("s") * NUM_CORES + lax.axis_index("c")
        t0 = wid * tblk
        pltpu.sync_copy(x_hbm.at[pl.ds(t0, tblk)], xst)

        def fire(n):
            pltpu.async_copy(w_hbm.at[xst.at[n]], bufs.at[n % NBUF], gsem)

        for n in range(NBUF - 1):
            fire(n)

        @pl.loop(0, tblk)
        def _(n):
            s = n % NBUF
            pltpu.make_async_copy(w_hbm.at[pl.ds(0, seq)], bufs.at[s],
                                  gsem).wait()
            pltpu.async_copy(bufs.at[s], out_hbm.at[t0 + n], wsem)

            @pl.when(n + NBUF - 1 < tblk)
            def _():
                @pl.when(n >= 1)
                def _():
                    pltpu.make_async_copy(bufs.at[0], out_hbm.at[0],
                                          wsem).wait()

                fire(n + NBUF - 1)

        for _ in range(NBUF):
            pltpu.make_async_copy(bufs.at[0], out_hbm.at[0], wsem).wait()

    return k


def kernel(x, weight):
    rows, cols = x.shape
    return _make_kernel(rows, cols)(x.astype(jnp.int32), weight)


# speedup vs baseline: 1.0017x; 1.0017x over previous
import functools

import jax
import jax.numpy as jnp
from jax import lax
from jax.experimental import pallas as pl
from jax.experimental.pallas import tpu as pltpu
from jax.experimental.pallas import tpu_sc as plsc

NUM_CORES = 2
NUM_SUBCORES = 16
NUM_WORKERS = NUM_CORES * NUM_SUBCORES
CHUNK = 128
DIM = 64
NBUF = 8


def _make_kernel(n_flat: int):
    mesh = plsc.VectorSubcoreMesh(core_axis_name="c", subcore_axis_name="s")
    blk = n_flat // NUM_WORKERS
    n_chunks = blk // CHUNK

    @functools.partial(
        pl.kernel,
        out_type=jax.ShapeDtypeStruct((n_flat, DIM), jnp.float32),
        mesh=mesh,
        scratch_types=[
            pltpu.VMEM((blk,), jnp.int32),
            pltpu.VMEM((NBUF, CHUNK, DIM), jnp.float32),
            pltpu.SemaphoreType.DMA,
            pltpu.SemaphoreType.DMA,
        ],
        compiler_params=pltpu.CompilerParams(use_tc_tiling_on_sc=False),
    )
    def k(x_hbm, w_hbm, out_hbm, idx_v, bufs, gsem, wsem):
        wid = lax.axis_index("s") * NUM_CORES + lax.axis_index("c")
        base = wid * blk
        pltpu.sync_copy(x_hbm.at[pl.ds(base, blk)], idx_v)

        def fire(n):
            pltpu.async_copy(w_hbm.at[idx_v.at[pl.ds(n * CHUNK, CHUNK)]],
                             bufs.at[n % NBUF], gsem)

        for n in range(NBUF - 1):
            fire(n)

        @pl.loop(0, n_chunks)
        def _(n):
            s = n % NBUF
            pltpu.make_async_copy(w_hbm.at[pl.ds(0, CHUNK)], bufs.at[s],
                                  gsem).wait()
            pltpu.async_copy(bufs.at[s],
                             out_hbm.at[pl.ds(base + n * CHUNK, CHUNK)], wsem)

            @pl.when(n + NBUF - 1 < n_chunks)
            def _():
                @pl.when(n >= 1)
                def _():
                    pltpu.make_async_copy(bufs.at[0],
                                          out_hbm.at[pl.ds(0, CHUNK)],
                                          wsem).wait()

                fire(n + NBUF - 1)

        for _ in range(NBUF):
            pltpu.make_async_copy(bufs.at[0], out_hbm.at[pl.ds(0, CHUNK)],
                                  wsem).wait()

    return k


def kernel(x, weight):
    rows, cols = x.shape
    xflat = x.reshape(rows * cols).astype(jnp.int32)
    out = _make_kernel(rows * cols)(xflat, weight)
    return out.reshape(rows, cols, DIM)
